# baseline (speedup 1.0000x reference)
import jax
import jax.numpy as jnp
from jax import lax
from jax.experimental import pallas as pl
from jax.experimental.pallas import tpu as pltpu
from jax.experimental.pallas import tpu_sc as plsc

N = 10000
NIN = 128
FVOL = 27
GROWS = N * FVOL

NC = 2
NT = 16
NW = NC * NT

RB = 512
LOGRB = 9
NBINP = 544
SLOT = NBINP // NW
OUTROWS = NBINP * RB
TBLW = NW * SLOT * 16

EB = 2000
BUF = 128
SCAP = 14336
SROWS = SCAP // 8
RCH = 128
NRCH = SROWS // RCH


def _iota16():
    return lax.broadcasted_iota(jnp.int32, (16,), 0)


def _bin_body(src_hbm, dst_hbm, off_hbm, binned_hbm, table_hbm,
              dstv, offv, srcv, hist, locp, goff, sortv, rowmap, histall,
              tbl, hist_spm, sem):
    cid = lax.axis_index("c")
    sid = lax.axis_index("s")
    e = src_hbm.shape[0]
    half = e // NC
    halfcap = half + 8 * NBINP * NT
    ept = half // NT
    ebase = cid * half + sid * ept
    nblk = ept // EB
    brows = binned_hbm.shape[0]

    iota = _iota16()
    zero16 = jnp.zeros((16,), jnp.int32)
    one16 = zero16 + 1
    rank0, _ = plsc.scan_count(zero16)
    basis = rank0 - iota

    for k in range(NBINP // 16):
        hist[pl.ds(k * 16, 16)] = zero16
    for k in range(1024 // 16):
        goff[pl.ds(k * 16, 16)] = zero16

    def blk_a(blk, carry):
        base = ebase + blk * EB
        pltpu.sync_copy(dst_hbm.at[pl.ds(base, EB)], dstv)
        pltpu.sync_copy(off_hbm.at[pl.ds(base, EB)], offv)

        def step(i, c):
            d = dstv[pl.ds(i * 16, 16)]
            o = offv[pl.ds(i * 16, 16)]
            b = lax.shift_right_logical(d * FVOL + o, LOGRB)
            cur = plsc.load_gather(hist, [b])
            rank, lastm = plsc.scan_count(b)
            plsc.store_scatter(hist, [b], cur + rank - basis + 1, mask=lastm)
            return c
        return lax.fori_loop(0, EB // 16, step, carry)
    lax.fori_loop(0, nblk, blk_a, 0)

    pltpu.sync_copy(hist, hist_spm.at[sid])
    plsc.subcore_barrier()
    pltpu.sync_copy(hist_spm, histall)

    core_base = cid * halfcap
    carry_l = jnp.int32(0)
    carry_g = jnp.int32(0)
    for grp in range(NBINP // 16):
        own = hist[pl.ds(grp * 16, 16)]
        own_p = jnp.bitwise_and(own + 7, ~7)
        tot_p = zero16
        below_p = zero16
        for t in range(NT):
            h = histall[t, pl.ds(grp * 16, 16)]
            hp = jnp.bitwise_and(h + 7, ~7)
            tot_p = tot_p + hp
            below_p = below_p + hp * jnp.where(sid > t, 1, 0).astype(jnp.int32)
        inc_l = plsc.cumsum(own_p)
        excl_l = inc_l - own_p + carry_l
        locp[pl.ds(grp * 16, 16)] = excl_l
        carry_l = carry_l + jnp.sum(own_p)

        inc_g = plsc.cumsum(tot_p)
        excl_g = inc_g - tot_p + carry_g
        gpos = excl_g + below_p + core_base
        goff[pl.ds(grp * 16, 16)] = lax.shift_right_logical(gpos, 3) - \
            lax.shift_right_logical(excl_l, 3)
        carry_g = carry_g + jnp.sum(tot_p)

        @pl.when(sid == 0)
        def _(grp=grp, excl_g=excl_g, tot_p=tot_p):
            bvec = iota + grp * 16
            w16 = jnp.bitwise_and(bvec, NW - 1)
            s16 = lax.shift_right_logical(bvec, 5)
            pos = (w16 * SLOT + s16) * 16
            plsc.store_scatter(tbl, [pos], excl_g + core_base)
            plsc.store_scatter(tbl, [pos + 1], tot_p)

    @pl.when(sid == 0)
    def _():
        pltpu.sync_copy(tbl, table_hbm.at[pl.ds(cid * TBLW, TBLW)])

    def prefill(grp, c):
        own = hist[pl.ds(grp * 16, 16)]
        own_p = jnp.bitwise_and(own + 7, ~7)
        lo = locp[pl.ds(grp * 16, 16)]
        gpad = (iota + grp * 16) * RB

        def pfill(p, c2):
            idx = lo + own + p
            m = (own + p) < own_p
            plsc.store_scatter(sortv,
                               [lax.shift_right_logical(idx, 3),
                                jnp.bitwise_and(idx, 7) * 2],
                               zero16 + N, mask=m)
            plsc.store_scatter(sortv,
                               [lax.shift_right_logical(idx, 3),
                                jnp.bitwise_and(idx, 7) * 2 + 1],
                               gpad, mask=m)
            return c2
        return lax.fori_loop(0, 7, pfill, c)
    lax.fori_loop(0, NBINP // 16, prefill, 0)

    def blk_b(blk, c):
        base = ebase + blk * EB
        pltpu.sync_copy(src_hbm.at[pl.ds(base, EB)], srcv)
        pltpu.sync_copy(dst_hbm.at[pl.ds(base, EB)], dstv)
        pltpu.sync_copy(off_hbm.at[pl.ds(base, EB)], offv)

        def step(i, c2):
            s = srcv[pl.ds(i * 16, 16)]
            d = dstv[pl.ds(i * 16, 16)]
            o = offv[pl.ds(i * 16, 16)]
            g = d * FVOL + o
            b = lax.shift_right_logical(g, LOGRB)
            cur = plsc.load_gather(locp, [b])
            rank, lastm = plsc.scan_count(b)
            pos = cur + rank - basis
            plsc.store_scatter(locp, [b], pos + 1, mask=lastm)
            r8 = lax.shift_right_logical(pos, 3)
            c8 = jnp.bitwise_and(pos, 7) * 2
            plsc.store_scatter(sortv, [r8, c8], s)
            plsc.store_scatter(sortv, [r8, c8 + 1], g)
            return c2
        return lax.fori_loop(0, EB // 16, step, c)
    lax.fori_loop(0, nblk, blk_b, 0)

    used_rows = lax.shift_right_logical(carry_l, 3)

    def rmap(j, c):
        r16 = iota + j * 16
        gl = plsc.load_gather(sortv, [r16, one16])
        b16 = jnp.bitwise_and(lax.shift_right_logical(gl, LOGRB), 1023)
        dif = plsc.load_gather(goff, [b16])
        gr = jnp.where(r16 < used_rows, r16 + dif, brows - 1)
        plsc.store_scatter(rowmap,
                           [lax.shift_right_logical(r16, 7),
                            jnp.bitwise_and(r16, 127)], gr)
        return c
    lax.fori_loop(0, SROWS // 16, rmap, 0)

    descs = []
    for j in range(NRCH):
        descs.append(pltpu.async_copy(
            sortv.at[pl.ds(j * RCH, RCH)],
            binned_hbm.at[rowmap.at[j]], sem))
    for d in descs:
        d.wait()


def _acc_body(x_hbm, binned_hbm, table_hbm, out_hbm,
              stge, srcbuf, gbuf, rowbuf, accum, tblv, sem):
    cid = lax.axis_index("c")
    sid = lax.axis_index("s")
    w = sid * NC + cid

    iota = _iota16()
    zero16 = jnp.zeros((16,), jnp.int32)
    one16 = zero16 + 1
    zrow = jnp.zeros((16,), jnp.float32)

    pltpu.sync_copy(table_hbm.at[pl.ds(w * SLOT * 16, SLOT * 16)],
                    tblv.at[pl.ds(0, SLOT * 16)])
    pltpu.sync_copy(table_hbm.at[pl.ds(TBLW + w * SLOT * 16, SLOT * 16)],
                    tblv.at[pl.ds(SLOT * 16, SLOT * 16)])

    def slot_body(k, _):
        bbin = k * NW + w

        def zacc(r, c):
            for u in range(NIN // 16):
                accum[r, pl.ds(u * 16, 16)] = zrow
            return c
        lax.fori_loop(0, RB + 1, zacc, 0)

        for c in range(NC):
            v = tblv[pl.ds((c * SLOT + k) * 16, 16)]
            start = v[0]
            seglen = v[1]
            nbatch = lax.shift_right_logical(seglen + (BUF - 1), 7)

            def batch(j, carry, start=start, seglen=seglen):
                bstart = pl.multiple_of(start + j * BUF, 8)
                valid = seglen - j * BUF
                pltpu.sync_copy(binned_hbm.at[pl.ds(bstart, BUF)], stge)

                def sub(t, cc):
                    idx = iota + t * 16
                    m = idx < valid
                    srcs = plsc.load_gather(stge, [idx, zero16])
                    gs = plsc.load_gather(stge, [idx, one16])
                    gl = jnp.where(m, jnp.bitwise_and(gs, RB - 1), RB)
                    ss = jnp.where(m, srcs, 0)
                    srcbuf[pl.ds(t * 16, 16)] = ss
                    gbuf[pl.ds(t * 16, 16)] = gl
                    return cc
                lax.fori_loop(0, BUF // 16, sub, 0)

                pltpu.async_copy(x_hbm.at[srcbuf], rowbuf, sem).wait()

                def grp16(r, cc):
                    rows = iota + r * 16
                    g16 = gbuf[pl.ds(r * 16, 16)]

                    def col(ci, c2):
                        for u in range(8):
                            csp = zero16 + (ci * 8 + u)
                            vals = plsc.load_gather(rowbuf, [rows, csp])
                            plsc.addupdate_scatter(accum, [g16, csp], vals)
                        return c2
                    return lax.fori_loop(0, NIN // 8, col, cc)
                lax.fori_loop(0, BUF // 16, grp16, 0)
                return carry
            lax.fori_loop(0, nbatch, batch, 0)

        pltpu.sync_copy(accum.at[pl.ds(0, RB)],
                        out_hbm.at[pl.ds(bbin * RB, RB)])
        return _
    lax.fori_loop(0, SLOT, slot_body, 0)


@jax.jit
def _run(x, src, dst, off):
    mesh = plsc.VectorSubcoreMesh(core_axis_name="c", subcore_axis_name="s")
    cparams = pltpu.CompilerParams(needs_layout_passes=False)
    cparams_k1 = pltpu.CompilerParams(needs_layout_passes=False,
                                      use_tc_tiling_on_sc=False)
    e = src.shape[0]
    brows = (e + NC * 8 * NBINP * NT) // 8 + 16

    binned, table = pl.kernel(
        _bin_body,
        out_type=(
            jax.ShapeDtypeStruct((brows, 16), jnp.int32),
            jax.ShapeDtypeStruct((NC * TBLW,), jnp.int32),
        ),
        mesh=mesh,
        compiler_params=cparams_k1,
        scratch_types=[
            pltpu.VMEM((EB,), jnp.int32),
            pltpu.VMEM((EB,), jnp.int32),
            pltpu.VMEM((EB,), jnp.int32),
            pltpu.VMEM((NBINP,), jnp.int32),
            pltpu.VMEM((NBINP,), jnp.int32),
            pltpu.VMEM((1024,), jnp.int32),
            pltpu.VMEM((SROWS, 16), jnp.int32),
            pltpu.VMEM((NRCH, RCH), jnp.int32),
            pltpu.VMEM((NT, NBINP), jnp.int32),
            pltpu.VMEM((TBLW,), jnp.int32),
            pltpu.VMEM_SHARED((NT, NBINP), jnp.int32),
            pltpu.SemaphoreType.DMA,
        ],
    )(src, dst, off)

    out = pl.kernel(
        _acc_body,
        out_type=jax.ShapeDtypeStruct((OUTROWS, NIN), jnp.float32),
        mesh=mesh,
        compiler_params=cparams,
        scratch_types=[
            pltpu.VMEM((BUF, 2), jnp.int32),
            pltpu.VMEM((BUF,), jnp.int32),
            pltpu.VMEM((BUF,), jnp.int32),
            pltpu.VMEM((BUF, NIN), jnp.float32),
            pltpu.VMEM((RB + 1, NIN), jnp.float32),
            pltpu.VMEM((NC * SLOT * 16,), jnp.int32),
            pltpu.SemaphoreType.DMA,
        ],
    )(x, binned.reshape(brows * 8, 2), table)
    return out


def kernel(x, edge_index, edge_offset, weight):
    del weight
    src = edge_index[0].astype(jnp.int32)
    dst = edge_index[1].astype(jnp.int32)
    off = edge_offset.astype(jnp.int32)

    x = jnp.concatenate([x, jnp.zeros((1, NIN), x.dtype)])

    e = src.shape[0]
    epad = -(-e // (NW * EB)) * (NW * EB)
    if epad != e:
        pad = epad - e
        src = jnp.pad(src, (0, pad), constant_values=N)
        dst = jnp.pad(dst, (0, pad), constant_values=N - 1)
        off = jnp.pad(off, (0, pad), constant_values=FVOL - 1)

    out = _run(x, src, dst, off)
    return out[:GROWS].reshape(N, FVOL * NIN)


# speedup vs baseline: 1.1224x; 1.1224x over previous
import jax
import jax.numpy as jnp
from jax import lax
from jax.experimental import pallas as pl
from jax.experimental.pallas import tpu as pltpu
from jax.experimental.pallas import tpu_sc as plsc

N = 10000
NIN = 128
FVOL = 27
GROWS = N * FVOL

NC = 2
NT = 16
NW = NC * NT

RB = 512
LOGRB = 9
NBINP = 544
SLOT = NBINP // NW
OUTROWS = NBINP * RB
TBLW = NW * SLOT * 16

EB = 2000
BUF = 128
SCAP = 14336
SROWS = SCAP // 8
RCH = 128
NRCH = SROWS // RCH


def _iota16():
    return lax.broadcasted_iota(jnp.int32, (16,), 0)


def _bin_body(src_hbm, dst_hbm, off_hbm, binned_hbm, table_hbm,
              dstv, offv, srcv, hist, locp, goff, sortv, rowmap, histall,
              tbl, hist_spm, sem):
    cid = lax.axis_index("c")
    sid = lax.axis_index("s")
    e = src_hbm.shape[0]
    half = e // NC
    halfcap = half + 8 * NBINP * NT
    ept = half // NT
    ebase = cid * half + sid * ept
    nblk = ept // EB
    brows = binned_hbm.shape[0]

    iota = _iota16()
    zero16 = jnp.zeros((16,), jnp.int32)
    one16 = zero16 + 1
    rank0, _ = plsc.scan_count(zero16)
    basis = rank0 - iota

    for k in range(NBINP // 16):
        hist[pl.ds(k * 16, 16)] = zero16
    for k in range(1024 // 16):
        goff[pl.ds(k * 16, 16)] = zero16

    def blk_a(blk, carry):
        base = ebase + blk * EB
        pltpu.sync_copy(dst_hbm.at[pl.ds(base, EB)], dstv)
        pltpu.sync_copy(off_hbm.at[pl.ds(base, EB)], offv)

        def step(i, c):
            d = dstv[pl.ds(i * 16, 16)]
            o = offv[pl.ds(i * 16, 16)]
            b = lax.shift_right_logical(d * FVOL + o, LOGRB)
            cur = plsc.load_gather(hist, [b])
            rank, lastm = plsc.scan_count(b)
            plsc.store_scatter(hist, [b], cur + rank - basis + 1, mask=lastm)
            return c
        return lax.fori_loop(0, EB // 16, step, carry)
    lax.fori_loop(0, nblk, blk_a, 0)

    pltpu.sync_copy(hist, hist_spm.at[sid])
    plsc.subcore_barrier()
    pltpu.sync_copy(hist_spm, histall)

    core_base = cid * halfcap
    carry_l = jnp.int32(0)
    carry_g = jnp.int32(0)
    for grp in range(NBINP // 16):
        own = hist[pl.ds(grp * 16, 16)]
        own_p = jnp.bitwise_and(own + 7, ~7)
        tot_p = zero16
        below_p = zero16
        for t in range(NT):
            h = histall[t, pl.ds(grp * 16, 16)]
            hp = jnp.bitwise_and(h + 7, ~7)
            tot_p = tot_p + hp
            below_p = below_p + hp * jnp.where(sid > t, 1, 0).astype(jnp.int32)
        inc_l = plsc.cumsum(own_p)
        excl_l = inc_l - own_p + carry_l
        locp[pl.ds(grp * 16, 16)] = excl_l
        carry_l = carry_l + jnp.sum(own_p)

        inc_g = plsc.cumsum(tot_p)
        excl_g = inc_g - tot_p + carry_g
        gpos = excl_g + below_p + core_base
        goff[pl.ds(grp * 16, 16)] = lax.shift_right_logical(gpos, 3) - \
            lax.shift_right_logical(excl_l, 3)
        carry_g = carry_g + jnp.sum(tot_p)

        @pl.when(sid == 0)
        def _(grp=grp, excl_g=excl_g, tot_p=tot_p):
            bvec = iota + grp * 16
            w16 = jnp.bitwise_and(bvec, NW - 1)
            s16 = lax.shift_right_logical(bvec, 5)
            pos = (w16 * SLOT + s16) * 16
            plsc.store_scatter(tbl, [pos], excl_g + core_base)
            plsc.store_scatter(tbl, [pos + 1], tot_p)

    @pl.when(sid == 0)
    def _():
        pltpu.sync_copy(tbl, table_hbm.at[pl.ds(cid * TBLW, TBLW)])

    def prefill(grp, c):
        own = hist[pl.ds(grp * 16, 16)]
        own_p = jnp.bitwise_and(own + 7, ~7)
        lo = locp[pl.ds(grp * 16, 16)]
        gpad = (iota + grp * 16) * RB

        def pfill(p, c2):
            idx = lo + own + p
            m = (own + p) < own_p
            plsc.store_scatter(sortv,
                               [lax.shift_right_logical(idx, 3),
                                jnp.bitwise_and(idx, 7) * 2],
                               zero16 + N, mask=m)
            plsc.store_scatter(sortv,
                               [lax.shift_right_logical(idx, 3),
                                jnp.bitwise_and(idx, 7) * 2 + 1],
                               gpad, mask=m)
            return c2
        return lax.fori_loop(0, 7, pfill, c)
    lax.fori_loop(0, NBINP // 16, prefill, 0)

    def blk_b(blk, c):
        base = ebase + blk * EB
        pltpu.sync_copy(src_hbm.at[pl.ds(base, EB)], srcv)
        pltpu.sync_copy(dst_hbm.at[pl.ds(base, EB)], dstv)
        pltpu.sync_copy(off_hbm.at[pl.ds(base, EB)], offv)

        def step(i, c2):
            s = srcv[pl.ds(i * 16, 16)]
            d = dstv[pl.ds(i * 16, 16)]
            o = offv[pl.ds(i * 16, 16)]
            g = d * FVOL + o
            b = lax.shift_right_logical(g, LOGRB)
            cur = plsc.load_gather(locp, [b])
            rank, lastm = plsc.scan_count(b)
            pos = cur + rank - basis
            plsc.store_scatter(locp, [b], pos + 1, mask=lastm)
            r8 = lax.shift_right_logical(pos, 3)
            c8 = jnp.bitwise_and(pos, 7) * 2
            plsc.store_scatter(sortv, [r8, c8], s)
            plsc.store_scatter(sortv, [r8, c8 + 1], g)
            return c2
        return lax.fori_loop(0, EB // 16, step, c)
    lax.fori_loop(0, nblk, blk_b, 0)

    used_rows = lax.shift_right_logical(carry_l, 3)

    def rmap(j, c):
        r16 = iota + j * 16
        gl = plsc.load_gather(sortv, [r16, one16])
        b16 = jnp.bitwise_and(lax.shift_right_logical(gl, LOGRB), 1023)
        dif = plsc.load_gather(goff, [b16])
        gr = jnp.where(r16 < used_rows, r16 + dif, brows - 1)
        plsc.store_scatter(rowmap,
                           [lax.shift_right_logical(r16, 7),
                            jnp.bitwise_and(r16, 127)], gr)
        return c
    lax.fori_loop(0, SROWS // 16, rmap, 0)

    descs = []
    for j in range(NRCH):
        descs.append(pltpu.async_copy(
            sortv.at[pl.ds(j * RCH, RCH)],
            binned_hbm.at[rowmap.at[j]], sem))
    for d in descs:
        d.wait()


def _acc_body(x_hbm, binned_hbm, table_hbm, out_hbm,
              stge, srcbuf, gbuf, rowbuf, accum, tblv, sem):
    cid = lax.axis_index("c")
    sid = lax.axis_index("s")
    w = sid * NC + cid

    iota = _iota16()
    zero16 = jnp.zeros((16,), jnp.int32)
    one16 = zero16 + 1
    zrow = jnp.zeros((16,), jnp.float32)

    pltpu.sync_copy(table_hbm.at[pl.ds(w * SLOT * 16, SLOT * 16)],
                    tblv.at[pl.ds(0, SLOT * 16)])
    pltpu.sync_copy(table_hbm.at[pl.ds(TBLW + w * SLOT * 16, SLOT * 16)],
                    tblv.at[pl.ds(SLOT * 16, SLOT * 16)])

    def slot_body(k, _):
        bbin = k * NW + w

        def zacc(r, c):
            for u in range(NIN // 16):
                accum[r, pl.ds(u * 16, 16)] = zrow
            return c
        lax.fori_loop(0, RB + 1, zacc, 0)

        for c in range(NC):
            v = tblv[pl.ds((c * SLOT + k) * 16, 16)]
            start = v[0]
            seglen = v[1]
            nbatch = lax.shift_right_logical(seglen + (BUF - 1), 7)

            def batch(j, carry, start=start, seglen=seglen):
                bstart = pl.multiple_of(start + j * BUF, 8)
                valid = seglen - j * BUF
                pltpu.sync_copy(binned_hbm.at[pl.ds(bstart, BUF)], stge)

                def sub(t, cc):
                    idx = iota + t * 16
                    m = idx < valid
                    srcs = plsc.load_gather(stge, [idx, zero16])
                    gs = plsc.load_gather(stge, [idx, one16])
                    gl = jnp.where(m, jnp.bitwise_and(gs, RB - 1), RB)
                    ss = jnp.where(m, srcs, 0)
                    srcbuf[pl.ds(t * 16, 16)] = ss
                    gbuf[pl.ds(t * 16, 16)] = gl
                    return cc
                lax.fori_loop(0, BUF // 16, sub, 0)

                pltpu.async_copy(x_hbm.at[srcbuf], rowbuf, sem).wait()

                def grp16(r, cc):
                    rows = iota + r * 16
                    g16 = gbuf[pl.ds(r * 16, 16)]

                    def col(ci, c2):
                        for u in range(8):
                            csp = jnp.bitwise_and(iota + (ci * 8 + u),
                                                  NIN - 1)
                            vals = plsc.load_gather(rowbuf, [rows, csp])
                            plsc.addupdate_scatter(accum, [g16, csp], vals)
                        return c2
                    return lax.fori_loop(0, NIN // 8, col, cc)
                lax.fori_loop(0, BUF // 16, grp16, 0)
                return carry
            lax.fori_loop(0, nbatch, batch, 0)

        pltpu.sync_copy(accum.at[pl.ds(0, RB)],
                        out_hbm.at[pl.ds(bbin * RB, RB)])
        return _
    lax.fori_loop(0, SLOT, slot_body, 0)


@jax.jit
def _run(x, src, dst, off):
    mesh = plsc.VectorSubcoreMesh(core_axis_name="c", subcore_axis_name="s")
    cparams = pltpu.CompilerParams(needs_layout_passes=False)
    cparams_k1 = pltpu.CompilerParams(needs_layout_passes=False,
                                      use_tc_tiling_on_sc=False)
    e = src.shape[0]
    brows = (e + NC * 8 * NBINP * NT) // 8 + 16

    binned, table = pl.kernel(
        _bin_body,
        out_type=(
            jax.ShapeDtypeStruct((brows, 16), jnp.int32),
            jax.ShapeDtypeStruct((NC * TBLW,), jnp.int32),
        ),
        mesh=mesh,
        compiler_params=cparams_k1,
        scratch_types=[
            pltpu.VMEM((EB,), jnp.int32),
            pltpu.VMEM((EB,), jnp.int32),
            pltpu.VMEM((EB,), jnp.int32),
            pltpu.VMEM((NBINP,), jnp.int32),
            pltpu.VMEM((NBINP,), jnp.int32),
            pltpu.VMEM((1024,), jnp.int32),
            pltpu.VMEM((SROWS, 16), jnp.int32),
            pltpu.VMEM((NRCH, RCH), jnp.int32),
            pltpu.VMEM((NT, NBINP), jnp.int32),
            pltpu.VMEM((TBLW,), jnp.int32),
            pltpu.VMEM_SHARED((NT, NBINP), jnp.int32),
            pltpu.SemaphoreType.DMA,
        ],
    )(src, dst, off)

    out = pl.kernel(
        _acc_body,
        out_type=jax.ShapeDtypeStruct((OUTROWS, NIN), jnp.float32),
        mesh=mesh,
        compiler_params=cparams,
        scratch_types=[
            pltpu.VMEM((BUF, 2), jnp.int32),
            pltpu.VMEM((BUF,), jnp.int32),
            pltpu.VMEM((BUF,), jnp.int32),
            pltpu.VMEM((BUF, NIN), jnp.float32),
            pltpu.VMEM((RB + 1, NIN), jnp.float32),
            pltpu.VMEM((NC * SLOT * 16,), jnp.int32),
            pltpu.SemaphoreType.DMA,
        ],
    )(x, binned.reshape(brows * 8, 2), table)
    return out


def kernel(x, edge_index, edge_offset, weight):
    del weight
    src = edge_index[0].astype(jnp.int32)
    dst = edge_index[1].astype(jnp.int32)
    off = edge_offset.astype(jnp.int32)

    x = jnp.concatenate([x, jnp.zeros((1, NIN), x.dtype)])

    e = src.shape[0]
    epad = -(-e // (NW * EB)) * (NW * EB)
    if epad != e:
        pad = epad - e
        src = jnp.pad(src, (0, pad), constant_values=N)
        dst = jnp.pad(dst, (0, pad), constant_values=N - 1)
        off = jnp.pad(off, (0, pad), constant_values=FVOL - 1)

    out = _run(x, src, dst, off)
    return out[:GROWS].reshape(N, FVOL * NIN)


# speedup vs baseline: 1.1781x; 1.0496x over previous
import functools

import jax
import jax.numpy as jnp
from jax import lax
from jax.experimental import pallas as pl
from jax.experimental.pallas import tpu as pltpu
from jax.experimental.pallas import tpu_sc as plsc

N = 10000
NIN = 128
FVOL = 27
GROWS = N * FVOL

NC = 2
NT = 16

CH = 13056
NCHUNK = -(-GROWS // CH)
NCHUNK += NCHUNK % 2
CPC = NCHUNK // NC
OUTROWS = NCHUNK * CH
TRASH = CH
STRIPE = CH // NT

EB = 2000
BUF = 128
FLUSH_AT = BUF - 16


def _sc_body(x_hbm, src_hbm, dst_hbm, off_hbm, out_hbm,
             srcv, dstv, offv, srcbuf, gbuf, rowbuf, spmem, sem):
    cid = lax.axis_index("c")
    sid = lax.axis_index("s")
    ept = src_hbm.shape[0] // NT
    nblk = ept // EB
    ebase = sid * ept

    def zero_rowbuf():
        def zrow(r, carry):
            for k in range(NIN // 16):
                rowbuf[r, pl.ds(k * 16, 16)] = jnp.zeros((16,), jnp.float32)
            return carry
        lax.fori_loop(0, BUF, zrow, 0)

    def reset_bufs():
        for k in range(BUF // 16):
            srcbuf[pl.ds(k * 16, 16)] = jnp.zeros((16,), jnp.int32)
            gbuf[pl.ds(k * 16, 16)] = jnp.full((16,), TRASH, jnp.int32)

    def flush():
        pltpu.async_copy(x_hbm.at[srcbuf], rowbuf, sem).wait()
        pltpu.sync_copy(rowbuf, spmem.at[gbuf], add=True)
        reset_bufs()

    reset_bufs()

    def chunk_body(ci, _):
        lo = (cid * CPC + ci) * CH

        zero_rowbuf()
        sbase = sid * STRIPE
        nfull, rem = STRIPE // BUF, STRIPE % BUF
        for j in range(nfull):
            pltpu.sync_copy(rowbuf, spmem.at[pl.ds(sbase + j * BUF, BUF)])
        if rem:
            pltpu.sync_copy(rowbuf.at[pl.ds(0, rem)],
                            spmem.at[pl.ds(sbase + nfull * BUF, rem)])
        plsc.subcore_barrier()

        def blk_body(blk, cnt):
            base = ebase + blk * EB
            pltpu.sync_copy(src_hbm.at[pl.ds(base, EB)], srcv)
            pltpu.sync_copy(dst_hbm.at[pl.ds(base, EB)], dstv)
            pltpu.sync_copy(off_hbm.at[pl.ds(base, EB)], offv)

            def step(i, cnt):
                s16 = srcv[pl.ds(i * 16, 16)]
                d16 = dstv[pl.ds(i * 16, 16)]
                o16 = offv[pl.ds(i * 16, 16)]
                g = d16 * FVOL + o16 - lo
                m = (g >= 0) & (g < CH)
                scan = plsc.cumsum(m.astype(jnp.int32))
                pos = scan + (cnt - 1)
                plsc.store_scatter(srcbuf, [pos], s16, mask=m)
                plsc.store_scatter(gbuf, [pos], g, mask=m)
                cnt2 = cnt + jnp.sum(m.astype(jnp.int32))
                do_flush = cnt2 >= FLUSH_AT

                @pl.when(do_flush)
                def _():
                    flush()

                return jnp.where(do_flush, 0, cnt2)

            return lax.fori_loop(0, EB // 16, step, cnt)

        lax.fori_loop(0, nblk, blk_body, jnp.int32(0))
        flush()
        plsc.subcore_barrier()

        pltpu.sync_copy(spmem.at[pl.ds(sid * STRIPE, STRIPE)],
                        out_hbm.at[pl.ds(lo + sid * STRIPE, STRIPE)])
        plsc.subcore_barrier()
        return _

    lax.fori_loop(0, CPC, chunk_body, 0)


@functools.partial(jax.jit, static_argnames=())
def _run(x, src, dst, off):
    mesh = plsc.VectorSubcoreMesh(core_axis_name="c", subcore_axis_name="s")
    kcall = pl.kernel(
        _sc_body,
        out_type=jax.ShapeDtypeStruct((OUTROWS, NIN), jnp.float32),
        mesh=mesh,
        compiler_params=pltpu.CompilerParams(needs_layout_passes=False),
        scratch_types=[
            pltpu.VMEM((EB,), jnp.int32),
            pltpu.VMEM((EB,), jnp.int32),
            pltpu.VMEM((EB,), jnp.int32),
            pltpu.VMEM((BUF,), jnp.int32),
            pltpu.VMEM((BUF,), jnp.int32),
            pltpu.VMEM((BUF, NIN), jnp.float32),
            pltpu.VMEM_SHARED((CH + 1, NIN), jnp.float32),
            pltpu.SemaphoreType.DMA,
        ],
    )
    return kcall(x, src, dst, off)


def kernel(x, edge_index, edge_offset, weight):
    del weight
    src = edge_index[0].astype(jnp.int32)
    dst = edge_index[1].astype(jnp.int32)
    off = edge_offset.astype(jnp.int32)

    e = src.shape[0]
    epad = -(-e // (NT * EB)) * (NT * EB)
    if epad != e:
        pad = epad - e
        src = jnp.pad(src, (0, pad))
        dst = jnp.pad(dst, (0, pad))
        off = jnp.pad(off, (0, pad), constant_values=-1)

    out = _run(x, src, dst, off)
    return out[:GROWS].reshape(N, FVOL * NIN)
